# Initial kernel scaffold; baseline (speedup 1.0000x reference)
import jax
import jax.numpy as jnp
from jax.experimental import pallas as pl


def kernel(q, k, v):
    raise NotImplementedError("write your pallas kernel here")


# speedup vs baseline: 2.5017x; 2.5017x over previous
import jax
import jax.numpy as jnp
from jax.experimental import pallas as pl
from jax.experimental.pallas import tpu as pltpu

BM = 128
BN = 64
NKM = BM // 4
NKN = BN // 4
LN2 = 0.6931471805599453
NEG = -1e30


def _mm(a, b):
    return jax.lax.dot_general(a, b, (((1,), (0,)), ((), ())),
                               preferred_element_type=jnp.float32,
                               precision=jax.lax.Precision.HIGHEST)


def _mmT(a, b):
    return jax.lax.dot_general(a, b, (((1,), (1,)), ((), ())),
                               preferred_element_type=jnp.float32,
                               precision=jax.lax.Precision.HIGHEST)


def _mmd(a, b):
    return jax.lax.dot_general(a, b, (((1,), (0,)), ((), ())),
                               preferred_element_type=jnp.float32)


def _mmTd(a, b):
    return jax.lax.dot_general(a, b, (((1,), (1,)), ((), ())),
                               preferred_element_type=jnp.float32)


def _mask_kernel(q_ref, k_ref, selq_ref, selk_ref, ids_ref):
    L = q_ref.shape[1]
    nbq, nbk = L // BM, L // BN
    selq = selq_ref[0]
    selk = selk_ref[0]
    sq = jnp.concatenate(
        [_mm(selq, q_ref[0, i * BM:(i + 1) * BM, :]) for i in range(nbq)], 0)
    sk = jnp.concatenate(
        [_mm(selk, k_ref[0, j * BN:(j + 1) * BN, :]) for j in range(nbk)], 0)
    scale = 1.0 / (q_ref.shape[2] ** 0.5)
    logits = _mmTd(sq, sk) * scale
    m = jnp.max(logits, axis=1, keepdims=True)
    e = jnp.exp(logits - m)
    probs = e / jnp.sum(e, axis=1, keepdims=True)
    r = jax.lax.broadcasted_iota(jnp.int32, (nbq * NKM, nbk), 0)
    c = jax.lax.broadcasted_iota(jnp.int32, (nbq * NKM, nbk), 1)
    sumk = jnp.where(r // NKN == c, 1.0, 0.0)
    pk = _mm(probs, sumk)
    r2 = jax.lax.broadcasted_iota(jnp.int32, (nbq * NKM, nbq), 0)
    c2 = jax.lax.broadcasted_iota(jnp.int32, (nbq * NKM, nbq), 1)
    meanq = jnp.where(r2 // NKM == c2, 1.0 / NKM, 0.0)
    pooling = jax.lax.dot_general(meanq, pk, (((0,), (0,)), ((), ())),
                                  preferred_element_type=jnp.float32,
                                  precision=jax.lax.Precision.HIGHEST)
    col = jax.lax.broadcasted_iota(jnp.int32, (nbq, nbk), 1)
    rank = jnp.zeros((nbq, nbk), jnp.int32)
    for jp in range(nbk):
        cv = pooling[:, jp:jp + 1]
        rank += (cv > pooling).astype(jnp.int32)
        rank += ((cv == pooling) & (jp < col)).astype(jnp.int32)
    ids = jnp.zeros((nbq, nbk), jnp.int32)
    for j in range(nbk):
        ids += j * (rank[:, j:j + 1] == col).astype(jnp.int32)
    ids_ref[0] = ids


def _attn_kernel(ids_ref, q_ref, k_ref, v_ref, o_ref):
    h = pl.program_id(0)
    qb = pl.program_id(1)
    q = q_ref[0]
    D = q.shape[1]
    scale = 1.0 / (D ** 0.5)
    r8 = jax.lax.broadcasted_iota(jnp.int32, (8, BN), 0)
    c8 = jax.lax.broadcasted_iota(jnp.int32, (8, BN), 1)
    P8 = jnp.where(c8 // 8 == r8, 0.125, 0.0)
    r2 = jax.lax.broadcasted_iota(jnp.int32, (BN // 2, BN), 0)
    c2 = jax.lax.broadcasted_iota(jnp.int32, (BN // 2, BN), 1)
    P2 = jnp.where(c2 // 2 == r2, 0.5, 0.0)

    kbs = []
    vbs = []
    for s in range(12):
        bid = ids_ref[h, qb, s]
        kbs.append(k_ref[0, pl.ds(bid * BN, BN), :])
        vbs.append(v_ref[0, pl.ds(bid * BN, BN), :])

    segs = [_mmTd(q, kbs[s]) for s in range(4)]
    kp2 = _mm(P2, kbs[4])
    segs.append(_mmTd(q, kp2))
    ktail = jnp.concatenate(
        [_mm(P8, kbs[s]) for s in range(5, 12)] + [jnp.zeros((8, BN), jnp.float32)], 0)
    segs.append(_mmTd(q, ktail))
    s = jnp.concatenate(segs, 1) * scale
    ci = jax.lax.broadcasted_iota(jnp.int32, (1, 352), 1)
    bias = jnp.where(ci < 256, 0.0,
                     jnp.where(ci < 288, LN2,
                               jnp.where(ci < 344, 3.0 * LN2, NEG)))
    s = s + bias
    m = jnp.max(s, axis=1, keepdims=True)
    e = jnp.exp(s - m)
    l = jnp.sum(e, axis=1, keepdims=True)
    acc = _mmd(e[:, 0:64], vbs[0])
    for t in range(1, 4):
        acc += _mmd(e[:, 64 * t:64 * (t + 1)], vbs[t])
    acc += _mmd(e[:, 256:288], _mm(P2, vbs[4]))
    vtail = jnp.concatenate(
        [_mm(P8, vbs[s2]) for s2 in range(5, 12)] + [jnp.zeros((8, BN), jnp.float32)], 0)
    acc += _mmd(e[:, 288:352], vtail)
    o_ref[0] = acc / l


def _sample_onehots(B, H):
    key = jax.random.key(42)
    k1, k2 = jax.random.split(key)
    rvq = jax.random.uniform(k1, (B, H, 1, BM))
    _, idxq = jax.lax.top_k(rvq, NKM)
    rvk = jax.random.uniform(k2, (B, H, 1, BN))
    _, idxk = jax.lax.top_k(rvk, NKN)
    selq = jax.nn.one_hot(idxq[:, :, 0, :], BM, dtype=jnp.float32)
    selk = jax.nn.one_hot(idxk[:, :, 0, :], BN, dtype=jnp.float32)
    return selq.reshape(B * H, NKM, BM), selk.reshape(B * H, NKN, BN)


def kernel(q, k, v, interpret=False):
    B, H, L, D = q.shape
    BH = B * H
    q3 = q.reshape(BH, L, D)
    k3 = k.reshape(BH, L, D)
    v3 = v.reshape(BH, L, D)
    selq, selk = _sample_onehots(B, H)
    nbq, nbk = L // BM, L // BN

    ids = pl.pallas_call(
        _mask_kernel,
        grid=(BH,),
        in_specs=[
            pl.BlockSpec((1, L, D), lambda h: (h, 0, 0)),
            pl.BlockSpec((1, L, D), lambda h: (h, 0, 0)),
            pl.BlockSpec((1, NKM, BM), lambda h: (h, 0, 0)),
            pl.BlockSpec((1, NKN, BN), lambda h: (h, 0, 0)),
        ],
        out_specs=pl.BlockSpec((1, nbq, nbk), lambda h: (h, 0, 0)),
        out_shape=jax.ShapeDtypeStruct((BH, nbq, nbk), jnp.int32),
        interpret=interpret,
    )(q3, k3, selq, selk)

    grid_spec = pltpu.PrefetchScalarGridSpec(
        num_scalar_prefetch=1,
        grid=(BH, nbq),
        in_specs=[
            pl.BlockSpec((1, BM, D), lambda h, qb, ids_r: (h, qb, 0)),
            pl.BlockSpec((1, L, D), lambda h, qb, ids_r: (h, 0, 0)),
            pl.BlockSpec((1, L, D), lambda h, qb, ids_r: (h, 0, 0)),
        ],
        out_specs=pl.BlockSpec((1, BM, D), lambda h, qb, ids_r: (h, qb, 0)),
    )
    out = pl.pallas_call(
        _attn_kernel,
        grid_spec=grid_spec,
        out_shape=jax.ShapeDtypeStruct((BH, L, D), jnp.float32),
        interpret=interpret,
    )(ids[:, :, :12], q3, k3, v3)
    return out.reshape(B, H, L, D)


# speedup vs baseline: 3.5208x; 1.4074x over previous
import jax
import jax.numpy as jnp
from jax.experimental import pallas as pl
from jax.experimental.pallas import tpu as pltpu

BM = 128
BN = 64
NKM = BM // 4
NKN = BN // 4
LN2 = 0.6931471805599453
NEG = -1e30


def _mmx(a, b):
    return jax.lax.dot_general(a, b, (((1,), (0,)), ((), ())),
                               preferred_element_type=jnp.float32,
                               precision=jax.lax.Precision.HIGHEST)


def _mmd(a, b):
    return jax.lax.dot_general(a, b, (((1,), (0,)), ((), ())),
                               preferred_element_type=jnp.float32)


def _mmTd(a, b):
    return jax.lax.dot_general(a, b, (((1,), (1,)), ((), ())),
                               preferred_element_type=jnp.float32)


def _mask_kernel(q_ref, k_ref, selq_ref, selk_ref, ids_ref):
    L = q_ref.shape[2]
    nbq, nbk = L // BM, L // BN
    selq = selq_ref[0, 0]
    selk = selk_ref[0, 0]
    sq = jnp.concatenate(
        [_mmx(selq, q_ref[0, 0, i * BM:(i + 1) * BM, :]) for i in range(nbq)], 0)
    sk = jnp.concatenate(
        [_mmx(selk, k_ref[0, 0, j * BN:(j + 1) * BN, :]) for j in range(nbk)], 0)
    scale = 1.0 / (q_ref.shape[3] ** 0.5)
    logits = _mmTd(sq, sk) * scale
    m = jnp.max(logits, axis=1, keepdims=True)
    e = jnp.exp(logits - m)
    probs = e / jnp.sum(e, axis=1, keepdims=True)
    r = jax.lax.broadcasted_iota(jnp.int32, (nbq * NKM, nbk), 0)
    c = jax.lax.broadcasted_iota(jnp.int32, (nbq * NKM, nbk), 1)
    sumk = jnp.where(r // NKN == c, 1.0, 0.0)
    pk = _mmx(probs, sumk)
    r2 = jax.lax.broadcasted_iota(jnp.int32, (nbq * NKM, nbq), 0)
    c2 = jax.lax.broadcasted_iota(jnp.int32, (nbq * NKM, nbq), 1)
    meanq = jnp.where(r2 // NKM == c2, 1.0 / NKM, 0.0)
    pooling = jax.lax.dot_general(meanq, pk, (((0,), (0,)), ((), ())),
                                  preferred_element_type=jnp.float32,
                                  precision=jax.lax.Precision.HIGHEST)
    col = jax.lax.broadcasted_iota(jnp.int32, (nbq, nbk), 1)
    rank = jnp.zeros((nbq, nbk), jnp.int32)
    for jp in range(nbk):
        cv = pooling[:, jp:jp + 1]
        rank += (cv > pooling).astype(jnp.int32)
        rank += ((cv == pooling) & (jp < col)).astype(jnp.int32)
    ids = jnp.zeros((nbq, nbk), jnp.int32)
    for j in range(nbk):
        ids += j * (rank[:, j:j + 1] == col).astype(jnp.int32)
    ids_ref[0, 0] = ids


def _attn_kernel(ids_ref, q_ref, k_ref, v_ref, o_ref):
    b = pl.program_id(0)
    h = pl.program_id(1)
    qb = pl.program_id(2)
    q = q_ref[0, 0]
    D = q.shape[1]
    scale = 1.0 / (D ** 0.5)
    r8 = jax.lax.broadcasted_iota(jnp.int32, (8, BN), 0)
    c8 = jax.lax.broadcasted_iota(jnp.int32, (8, BN), 1)
    P8 = jnp.where(c8 // 8 == r8, 0.125, 0.0)
    r2 = jax.lax.broadcasted_iota(jnp.int32, (BN // 2, BN), 0)
    c2 = jax.lax.broadcasted_iota(jnp.int32, (BN // 2, BN), 1)
    P2 = jnp.where(c2 // 2 == r2, 0.5, 0.0)

    kbs = []
    vbs = []
    for s in range(12):
        bid = ids_ref[b, h, qb, s]
        kbs.append(k_ref[0, 0, pl.ds(bid * BN, BN), :])
        vbs.append(v_ref[0, 0, pl.ds(bid * BN, BN), :])

    zpad = jnp.zeros((40, BN), jnp.float32)
    ksel = jnp.concatenate(
        kbs[:4] + [_mmd(P2, kbs[4])] + [_mmd(P8, kbs[s]) for s in range(5, 12)]
        + [zpad], 0)
    vsel = jnp.concatenate(
        vbs[:4] + [_mmd(P2, vbs[4])] + [_mmd(P8, vbs[s]) for s in range(5, 12)]
        + [zpad], 0)

    s = _mmTd(q, ksel) * scale
    ci = jax.lax.broadcasted_iota(jnp.int32, (1, 384), 1)
    bias = jnp.where(ci < 256, 0.0,
                     jnp.where(ci < 288, LN2,
                               jnp.where(ci < 344, 3.0 * LN2, NEG)))
    s = s + bias
    m = jnp.max(s, axis=1, keepdims=True)
    e = jnp.exp(s - m)
    l = jnp.sum(e, axis=1, keepdims=True)
    o_ref[0, 0] = _mmd(e, vsel) / l


def _sample_onehots(B, H):
    key = jax.random.key(42)
    k1, k2 = jax.random.split(key)
    rvq = jax.random.uniform(k1, (B, H, 1, BM))
    _, idxq = jax.lax.top_k(rvq, NKM)
    rvk = jax.random.uniform(k2, (B, H, 1, BN))
    _, idxk = jax.lax.top_k(rvk, NKN)
    selq = jax.nn.one_hot(idxq[:, :, 0, :], BM, dtype=jnp.float32)
    selk = jax.nn.one_hot(idxk[:, :, 0, :], BN, dtype=jnp.float32)
    return selq, selk


def kernel(q, k, v, interpret=False):
    B, H, L, D = q.shape
    selq, selk = _sample_onehots(B, H)
    nbq, nbk = L // BM, L // BN

    ids = pl.pallas_call(
        _mask_kernel,
        grid=(B, H),
        in_specs=[
            pl.BlockSpec((1, 1, L, D), lambda b, h: (b, h, 0, 0)),
            pl.BlockSpec((1, 1, L, D), lambda b, h: (b, h, 0, 0)),
            pl.BlockSpec((1, 1, NKM, BM), lambda b, h: (b, h, 0, 0)),
            pl.BlockSpec((1, 1, NKN, BN), lambda b, h: (b, h, 0, 0)),
        ],
        out_specs=pl.BlockSpec((1, 1, nbq, nbk), lambda b, h: (b, h, 0, 0)),
        out_shape=jax.ShapeDtypeStruct((B, H, nbq, nbk), jnp.int32),
        interpret=interpret,
    )(q, k, selq, selk)

    grid_spec = pltpu.PrefetchScalarGridSpec(
        num_scalar_prefetch=1,
        grid=(B, H, nbq),
        in_specs=[
            pl.BlockSpec((1, 1, BM, D), lambda b, h, qb, ids_r: (b, h, qb, 0)),
            pl.BlockSpec((1, 1, L, D), lambda b, h, qb, ids_r: (b, h, 0, 0)),
            pl.BlockSpec((1, 1, L, D), lambda b, h, qb, ids_r: (b, h, 0, 0)),
        ],
        out_specs=pl.BlockSpec((1, 1, BM, D), lambda b, h, qb, ids_r: (b, h, qb, 0)),
    )
    out = pl.pallas_call(
        _attn_kernel,
        grid_spec=grid_spec,
        out_shape=jax.ShapeDtypeStruct((B, H, L, D), jnp.float32),
        interpret=interpret,
    )(ids, q, k, v)
    return out


# speedup vs baseline: 4.9481x; 1.4054x over previous
import jax
import jax.numpy as jnp
from jax.experimental import pallas as pl
from jax.experimental.pallas import tpu as pltpu

BM = 128
BN = 64
NKM = BM // 4
NKN = BN // 4
LN2 = 0.6931471805599453
NEG = -1e30


def _mmx(a, b):
    return jax.lax.dot_general(a, b, (((1,), (0,)), ((), ())),
                               preferred_element_type=jnp.float32,
                               precision=jax.lax.Precision.HIGHEST)


def _mmd(a, b):
    return jax.lax.dot_general(a, b, (((1,), (0,)), ((), ())),
                               preferred_element_type=jnp.float32)


def _mmTd(a, b):
    return jax.lax.dot_general(a, b, (((1,), (1,)), ((), ())),
                               preferred_element_type=jnp.float32)


def _mask_kernel(q_ref, k_ref, selq_ref, selk_ref, ids_ref):
    L = q_ref.shape[2]
    nbq, nbk = L // BM, L // BN
    selq = selq_ref[0, 0]
    selk = selk_ref[0, 0]
    sq = jnp.concatenate(
        [_mmd(selq, q_ref[0, 0, i * BM:(i + 1) * BM, :]) for i in range(nbq)], 0)
    sk = jnp.concatenate(
        [_mmd(selk, k_ref[0, 0, j * BN:(j + 1) * BN, :]) for j in range(nbk)], 0)
    scale = 1.0 / (q_ref.shape[3] ** 0.5)
    logits = _mmTd(sq, sk) * scale
    m = jnp.max(logits, axis=1, keepdims=True)
    e = jnp.exp(logits - m)
    probs = e / jnp.sum(e, axis=1, keepdims=True)
    r = jax.lax.broadcasted_iota(jnp.int32, (nbq * NKM, nbk), 0)
    c = jax.lax.broadcasted_iota(jnp.int32, (nbq * NKM, nbk), 1)
    sumk = jnp.where(r // NKN == c, 1.0, 0.0)
    p_hi = probs.astype(jnp.bfloat16).astype(jnp.float32)
    r1 = probs - p_hi
    p_mid = r1.astype(jnp.bfloat16).astype(jnp.float32)
    p_lo = r1 - p_mid
    pk = _mmd(p_hi, sumk) + _mmd(p_mid, sumk) + _mmd(p_lo, sumk)
    r2 = jax.lax.broadcasted_iota(jnp.int32, (nbq * NKM, nbq), 0)
    c2 = jax.lax.broadcasted_iota(jnp.int32, (nbq * NKM, nbq), 1)
    meanq = jnp.where(r2 // NKM == c2, 1.0 / NKM, 0.0)
    pooling = jax.lax.dot_general(meanq, pk, (((0,), (0,)), ((), ())),
                                  preferred_element_type=jnp.float32,
                                  precision=jax.lax.Precision.HIGHEST)
    col = jax.lax.broadcasted_iota(jnp.int32, (nbq, nbk), 1)
    rank = jnp.zeros((nbq, nbk), jnp.int32)
    for jp in range(nbk):
        cv = pooling[:, jp:jp + 1]
        rank += (cv > pooling).astype(jnp.int32)
        rank += ((cv == pooling) & (jp < col)).astype(jnp.int32)
    ids = jnp.zeros((nbq, nbk), jnp.int32)
    for j in range(nbk):
        ids += j * (rank[:, j:j + 1] == col).astype(jnp.int32)
    ids_ref[0, 0] = ids


QPG = 4


def _attn_kernel(ids_ref, q_ref, k_ref, v_ref, o_ref):
    b = pl.program_id(0)
    h = pl.program_id(1)
    qg = pl.program_id(2)
    D = q_ref.shape[3]
    scale = 1.0 / (D ** 0.5)
    r8 = jax.lax.broadcasted_iota(jnp.int32, (8, BN), 0)
    c8 = jax.lax.broadcasted_iota(jnp.int32, (8, BN), 1)
    P8 = jnp.where(c8 // 8 == r8, 0.125, 0.0)
    r2 = jax.lax.broadcasted_iota(jnp.int32, (BN // 2, BN), 0)
    c2 = jax.lax.broadcasted_iota(jnp.int32, (BN // 2, BN), 1)
    P2 = jnp.where(c2 // 2 == r2, 0.5, 0.0)
    ci = jax.lax.broadcasted_iota(jnp.int32, (1, 384), 1)
    bias = jnp.where(ci < 256, 0.0,
                     jnp.where(ci < 288, LN2,
                               jnp.where(ci < 344, 3.0 * LN2, NEG)))
    zpad = jnp.zeros((40, BN), jnp.float32)

    for qq in range(QPG):
        qb = qg * QPG + qq
        q = q_ref[0, 0, qq * BM:(qq + 1) * BM, :]
        kbs = []
        vbs = []
        for s in range(12):
            bid = ids_ref[b, h, qb, s]
            kbs.append(k_ref[0, 0, pl.ds(bid * BN, BN), :])
            vbs.append(v_ref[0, 0, pl.ds(bid * BN, BN), :])
        ksel = jnp.concatenate(
            kbs[:4] + [_mmd(P2, kbs[4])] + [_mmd(P8, kbs[s]) for s in range(5, 12)]
            + [zpad], 0)
        vsel = jnp.concatenate(
            vbs[:4] + [_mmd(P2, vbs[4])] + [_mmd(P8, vbs[s]) for s in range(5, 12)]
            + [zpad], 0)
        s = _mmTd(q, ksel) * scale + bias
        m = jnp.max(s, axis=1, keepdims=True)
        e = jnp.exp(s - m)
        l = jnp.sum(e, axis=1, keepdims=True)
        o_ref[0, 0, qq * BM:(qq + 1) * BM, :] = _mmd(e, vsel) / l


def _sample_onehots(B, H):
    key = jax.random.key(42)
    k1, k2 = jax.random.split(key)
    rvq = jax.random.uniform(k1, (B, H, 1, BM))
    _, idxq = jax.lax.top_k(rvq, NKM)
    rvk = jax.random.uniform(k2, (B, H, 1, BN))
    _, idxk = jax.lax.top_k(rvk, NKN)
    selq = jax.nn.one_hot(idxq[:, :, 0, :], BM, dtype=jnp.float32)
    selk = jax.nn.one_hot(idxk[:, :, 0, :], BN, dtype=jnp.float32)
    return selq, selk


def kernel(q, k, v, interpret=False):
    B, H, L, D = q.shape
    selq, selk = _sample_onehots(B, H)
    nbq, nbk = L // BM, L // BN

    ids = pl.pallas_call(
        _mask_kernel,
        grid=(B, H),
        in_specs=[
            pl.BlockSpec((1, 1, L, D), lambda b, h: (b, h, 0, 0)),
            pl.BlockSpec((1, 1, L, D), lambda b, h: (b, h, 0, 0)),
            pl.BlockSpec((1, 1, NKM, BM), lambda b, h: (b, h, 0, 0)),
            pl.BlockSpec((1, 1, NKN, BN), lambda b, h: (b, h, 0, 0)),
        ],
        out_specs=pl.BlockSpec((1, 1, nbq, nbk), lambda b, h: (b, h, 0, 0)),
        out_shape=jax.ShapeDtypeStruct((B, H, nbq, nbk), jnp.int32),
        interpret=interpret,
    )(q, k, selq, selk)

    grid_spec = pltpu.PrefetchScalarGridSpec(
        num_scalar_prefetch=1,
        grid=(B, H, nbq // QPG),
        in_specs=[
            pl.BlockSpec((1, 1, QPG * BM, D), lambda b, h, qg, ids_r: (b, h, qg, 0)),
            pl.BlockSpec((1, 1, L, D), lambda b, h, qg, ids_r: (b, h, 0, 0)),
            pl.BlockSpec((1, 1, L, D), lambda b, h, qg, ids_r: (b, h, 0, 0)),
        ],
        out_specs=pl.BlockSpec((1, 1, QPG * BM, D), lambda b, h, qg, ids_r: (b, h, qg, 0)),
    )
    out = pl.pallas_call(
        _attn_kernel,
        grid_spec=grid_spec,
        out_shape=jax.ShapeDtypeStruct((B, H, L, D), jnp.float32),
        interpret=interpret,
    )(ids, q, k, v)
    return out


# speedup vs baseline: 5.0000x; 1.0105x over previous
import jax
import jax.numpy as jnp
from jax.experimental import pallas as pl
from jax.experimental.pallas import tpu as pltpu

BM = 128
BN = 64
NKM = BM // 4
NKN = BN // 4
LN2 = 0.6931471805599453
NEG = -1e30


def _mmx(a, b):
    return jax.lax.dot_general(a, b, (((1,), (0,)), ((), ())),
                               preferred_element_type=jnp.float32,
                               precision=jax.lax.Precision.HIGHEST)


def _mmd(a, b):
    return jax.lax.dot_general(a, b, (((1,), (0,)), ((), ())),
                               preferred_element_type=jnp.float32)


def _mmTd(a, b):
    return jax.lax.dot_general(a, b, (((1,), (1,)), ((), ())),
                               preferred_element_type=jnp.float32)


def _mask_kernel(q_ref, k_ref, selq_ref, selk_ref, ids_ref):
    L = q_ref.shape[2]
    nbq, nbk = L // BM, L // BN
    selq = selq_ref[0, 0]
    selk = selk_ref[0, 0]
    sq = jnp.concatenate(
        [_mmd(selq, q_ref[0, 0, i * BM:(i + 1) * BM, :]) for i in range(nbq)], 0)
    sk = jnp.concatenate(
        [_mmd(selk, k_ref[0, 0, j * BN:(j + 1) * BN, :]) for j in range(nbk)], 0)
    scale = 1.0 / (q_ref.shape[3] ** 0.5)
    logits = _mmTd(sq, sk) * scale
    m = jnp.max(logits, axis=1, keepdims=True)
    e = jnp.exp(logits - m)
    probs = e / jnp.sum(e, axis=1, keepdims=True)
    r = jax.lax.broadcasted_iota(jnp.int32, (nbq * NKM, nbk), 0)
    c = jax.lax.broadcasted_iota(jnp.int32, (nbq * NKM, nbk), 1)
    sumk = jnp.where(r // NKN == c, 1.0, 0.0)
    p_hi = probs.astype(jnp.bfloat16).astype(jnp.float32)
    r1 = probs - p_hi
    p_mid = r1.astype(jnp.bfloat16).astype(jnp.float32)
    p_lo = r1 - p_mid
    pk = _mmd(p_hi, sumk) + _mmd(p_mid, sumk) + _mmd(p_lo, sumk)
    r2 = jax.lax.broadcasted_iota(jnp.int32, (nbq * NKM, nbq), 0)
    c2 = jax.lax.broadcasted_iota(jnp.int32, (nbq * NKM, nbq), 1)
    meanq = jnp.where(r2 // NKM == c2, 1.0 / NKM, 0.0)
    pooling = jax.lax.dot_general(meanq, pk, (((0,), (0,)), ((), ())),
                                  preferred_element_type=jnp.float32,
                                  precision=jax.lax.Precision.HIGHEST)
    col = jax.lax.broadcasted_iota(jnp.int32, (nbq, nbk), 1)
    rank = jnp.zeros((nbq, nbk), jnp.int32)
    for jp in range(nbk):
        cv = pooling[:, jp:jp + 1]
        rank += (cv > pooling).astype(jnp.int32)
        rank += ((cv == pooling) & (jp < col)).astype(jnp.int32)
    ids = jnp.zeros((nbq, nbk), jnp.int32)
    for j in range(nbk):
        ids += j * (rank[:, j:j + 1] == col).astype(jnp.int32)
    ids_ref[0, 0] = ids


QPG = 8


def _attn_kernel(ids_ref, q_ref, k_ref, v_ref, o_ref):
    b = pl.program_id(0)
    h = pl.program_id(1)
    qg = pl.program_id(2)
    D = q_ref.shape[3]
    scale = 1.0 / (D ** 0.5)
    r8 = jax.lax.broadcasted_iota(jnp.int32, (8, BN), 0)
    c8 = jax.lax.broadcasted_iota(jnp.int32, (8, BN), 1)
    P8 = jnp.where(c8 // 8 == r8, 0.125, 0.0)
    r2 = jax.lax.broadcasted_iota(jnp.int32, (BN // 2, BN), 0)
    c2 = jax.lax.broadcasted_iota(jnp.int32, (BN // 2, BN), 1)
    P2 = jnp.where(c2 // 2 == r2, 0.5, 0.0)
    ci = jax.lax.broadcasted_iota(jnp.int32, (1, 384), 1)
    bias = jnp.where(ci < 256, 0.0,
                     jnp.where(ci < 288, LN2,
                               jnp.where(ci < 344, 3.0 * LN2, NEG)))
    zpad = jnp.zeros((40, BN), jnp.float32)

    for qq in range(QPG):
        qb = qg * QPG + qq
        q = q_ref[0, 0, qq * BM:(qq + 1) * BM, :]
        kbs = []
        vbs = []
        for s in range(12):
            bid = ids_ref[b, h, qb, s]
            kbs.append(k_ref[0, 0, pl.ds(bid * BN, BN), :])
            vbs.append(v_ref[0, 0, pl.ds(bid * BN, BN), :])
        ksel = jnp.concatenate(
            kbs[:4] + [_mmd(P2, kbs[4])] + [_mmd(P8, kbs[s]) for s in range(5, 12)]
            + [zpad], 0)
        vsel = jnp.concatenate(
            vbs[:4] + [_mmd(P2, vbs[4])] + [_mmd(P8, vbs[s]) for s in range(5, 12)]
            + [zpad], 0)
        s = _mmTd(q, ksel) * scale + bias
        m = jnp.max(s, axis=1, keepdims=True)
        e = jnp.exp(s - m)
        l = jnp.sum(e, axis=1, keepdims=True)
        o_ref[0, 0, qq * BM:(qq + 1) * BM, :] = _mmd(e, vsel) / l


def _sample_onehots(B, H):
    key = jax.random.key(42)
    k1, k2 = jax.random.split(key)
    rvq = jax.random.uniform(k1, (B, H, 1, BM))
    _, idxq = jax.lax.top_k(rvq, NKM)
    rvk = jax.random.uniform(k2, (B, H, 1, BN))
    _, idxk = jax.lax.top_k(rvk, NKN)
    selq = jax.nn.one_hot(idxq[:, :, 0, :], BM, dtype=jnp.float32)
    selk = jax.nn.one_hot(idxk[:, :, 0, :], BN, dtype=jnp.float32)
    return selq, selk


def kernel(q, k, v, interpret=False):
    B, H, L, D = q.shape
    selq, selk = _sample_onehots(B, H)
    nbq, nbk = L // BM, L // BN

    ids = pl.pallas_call(
        _mask_kernel,
        grid=(B, H),
        in_specs=[
            pl.BlockSpec((1, 1, L, D), lambda b, h: (b, h, 0, 0)),
            pl.BlockSpec((1, 1, L, D), lambda b, h: (b, h, 0, 0)),
            pl.BlockSpec((1, 1, NKM, BM), lambda b, h: (b, h, 0, 0)),
            pl.BlockSpec((1, 1, NKN, BN), lambda b, h: (b, h, 0, 0)),
        ],
        out_specs=pl.BlockSpec((1, 1, nbq, nbk), lambda b, h: (b, h, 0, 0)),
        out_shape=jax.ShapeDtypeStruct((B, H, nbq, nbk), jnp.int32),
        interpret=interpret,
    )(q, k, selq, selk)

    grid_spec = pltpu.PrefetchScalarGridSpec(
        num_scalar_prefetch=1,
        grid=(B, H, nbq // QPG),
        in_specs=[
            pl.BlockSpec((1, 1, QPG * BM, D), lambda b, h, qg, ids_r: (b, h, qg, 0)),
            pl.BlockSpec((1, 1, L, D), lambda b, h, qg, ids_r: (b, h, 0, 0)),
            pl.BlockSpec((1, 1, L, D), lambda b, h, qg, ids_r: (b, h, 0, 0)),
        ],
        out_specs=pl.BlockSpec((1, 1, QPG * BM, D), lambda b, h, qg, ids_r: (b, h, qg, 0)),
    )
    out = pl.pallas_call(
        _attn_kernel,
        grid_spec=grid_spec,
        out_shape=jax.ShapeDtypeStruct((B, H, L, D), jnp.float32),
        interpret=interpret,
    )(ids, q, k, v)
    return out


# speedup vs baseline: 5.0586x; 1.0117x over previous
import jax
import jax.numpy as jnp
from jax.experimental import pallas as pl
from jax.experimental.pallas import tpu as pltpu

BM = 128
BN = 64
NKM = BM // 4
NKN = BN // 4
LN2 = 0.6931471805599453
NEG = -1e30


def _mmx(a, b):
    return jax.lax.dot_general(a, b, (((1,), (0,)), ((), ())),
                               preferred_element_type=jnp.float32,
                               precision=jax.lax.Precision.HIGHEST)


def _mmd(a, b):
    return jax.lax.dot_general(a, b, (((1,), (0,)), ((), ())),
                               preferred_element_type=jnp.float32)


def _mmTd(a, b):
    return jax.lax.dot_general(a, b, (((1,), (1,)), ((), ())),
                               preferred_element_type=jnp.float32)


def _mask_kernel(q_ref, k_ref, selq_ref, selk_ref, ids_ref):
    L = q_ref.shape[2]
    nbq, nbk = L // BM, L // BN
    selq = selq_ref[0, 0]
    selk = selk_ref[0, 0]
    sq = jnp.concatenate(
        [_mmd(selq, q_ref[0, 0, i * BM:(i + 1) * BM, :]) for i in range(nbq)], 0)
    sk = jnp.concatenate(
        [_mmd(selk, k_ref[0, 0, j * BN:(j + 1) * BN, :]) for j in range(nbk)], 0)
    scale = 1.0 / (q_ref.shape[3] ** 0.5)
    logits = _mmTd(sq, sk) * scale
    m = jnp.max(logits, axis=1, keepdims=True)
    e = jnp.exp(logits - m)
    probs = e / jnp.sum(e, axis=1, keepdims=True)
    r = jax.lax.broadcasted_iota(jnp.int32, (nbq * NKM, nbk), 0)
    c = jax.lax.broadcasted_iota(jnp.int32, (nbq * NKM, nbk), 1)
    sumk = jnp.where(r // NKN == c, 1.0, 0.0)
    p_hi = probs.astype(jnp.bfloat16).astype(jnp.float32)
    r1 = probs - p_hi
    p_mid = r1.astype(jnp.bfloat16).astype(jnp.float32)
    p_lo = r1 - p_mid
    pk = _mmd(p_hi, sumk) + _mmd(p_mid, sumk) + _mmd(p_lo, sumk)
    r2 = jax.lax.broadcasted_iota(jnp.int32, (nbq * NKM, nbq), 0)
    c2 = jax.lax.broadcasted_iota(jnp.int32, (nbq * NKM, nbq), 1)
    meanq = jnp.where(r2 // NKM == c2, 1.0 / NKM, 0.0)
    pooling = jax.lax.dot_general(meanq, pk, (((0,), (0,)), ((), ())),
                                  preferred_element_type=jnp.float32,
                                  precision=jax.lax.Precision.HIGHEST)
    col = jax.lax.broadcasted_iota(jnp.int32, (nbq, nbk), 1)
    rank = jnp.zeros((nbq, nbk), jnp.int32)
    for jp in range(nbk):
        cv = pooling[:, jp:jp + 1]
        rank += (cv > pooling).astype(jnp.int32)
        rank += ((cv == pooling) & (jp < col)).astype(jnp.int32)
    ids = jnp.zeros((nbq, nbk), jnp.int32)
    for j in range(nbk):
        ids += j * (rank[:, j:j + 1] == col).astype(jnp.int32)
    ids_ref[0, 0] = ids


QPG = 16


def _attn_kernel(ids_ref, q_ref, k_ref, v_ref, o_ref):
    b = pl.program_id(0)
    h = pl.program_id(1)
    qg = pl.program_id(2)
    D = q_ref.shape[3]
    scale = 1.0 / (D ** 0.5)
    r8 = jax.lax.broadcasted_iota(jnp.int32, (8, BN), 0)
    c8 = jax.lax.broadcasted_iota(jnp.int32, (8, BN), 1)
    P8 = jnp.where(c8 // 8 == r8, 0.125, 0.0)
    r2 = jax.lax.broadcasted_iota(jnp.int32, (BN // 2, BN), 0)
    c2 = jax.lax.broadcasted_iota(jnp.int32, (BN // 2, BN), 1)
    P2 = jnp.where(c2 // 2 == r2, 0.5, 0.0)
    ci = jax.lax.broadcasted_iota(jnp.int32, (1, 384), 1)
    bias = jnp.where(ci < 256, 0.0,
                     jnp.where(ci < 288, LN2,
                               jnp.where(ci < 344, 3.0 * LN2, NEG)))
    zpad = jnp.zeros((40, BN), jnp.float32)

    for qq in range(QPG):
        qb = qg * QPG + qq
        q = q_ref[0, 0, qq * BM:(qq + 1) * BM, :]
        kbs = []
        vbs = []
        for s in range(12):
            bid = ids_ref[b, h, qb, s]
            kbs.append(k_ref[0, 0, pl.ds(bid * BN, BN), :])
            vbs.append(v_ref[0, 0, pl.ds(bid * BN, BN), :])
        ksel = jnp.concatenate(
            kbs[:4] + [_mmd(P2, kbs[4])] + [_mmd(P8, kbs[s]) for s in range(5, 12)]
            + [zpad], 0)
        vsel = jnp.concatenate(
            vbs[:4] + [_mmd(P2, vbs[4])] + [_mmd(P8, vbs[s]) for s in range(5, 12)]
            + [zpad], 0)
        s = _mmTd(q, ksel) * scale + bias
        m = jnp.max(s, axis=1, keepdims=True)
        e = jnp.exp(s - m)
        l = jnp.sum(e, axis=1, keepdims=True)
        o_ref[0, 0, qq * BM:(qq + 1) * BM, :] = _mmd(e, vsel) / l


def _sample_onehots(B, H):
    key = jax.random.key(42)
    k1, k2 = jax.random.split(key)
    rvq = jax.random.uniform(k1, (B, H, 1, BM))
    _, idxq = jax.lax.top_k(rvq, NKM)
    rvk = jax.random.uniform(k2, (B, H, 1, BN))
    _, idxk = jax.lax.top_k(rvk, NKN)
    selq = jax.nn.one_hot(idxq[:, :, 0, :], BM, dtype=jnp.float32)
    selk = jax.nn.one_hot(idxk[:, :, 0, :], BN, dtype=jnp.float32)
    return selq, selk


def kernel(q, k, v, interpret=False):
    B, H, L, D = q.shape
    selq, selk = _sample_onehots(B, H)
    nbq, nbk = L // BM, L // BN

    ids = pl.pallas_call(
        _mask_kernel,
        grid=(B, H),
        in_specs=[
            pl.BlockSpec((1, 1, L, D), lambda b, h: (b, h, 0, 0)),
            pl.BlockSpec((1, 1, L, D), lambda b, h: (b, h, 0, 0)),
            pl.BlockSpec((1, 1, NKM, BM), lambda b, h: (b, h, 0, 0)),
            pl.BlockSpec((1, 1, NKN, BN), lambda b, h: (b, h, 0, 0)),
        ],
        out_specs=pl.BlockSpec((1, 1, nbq, nbk), lambda b, h: (b, h, 0, 0)),
        out_shape=jax.ShapeDtypeStruct((B, H, nbq, nbk), jnp.int32),
        interpret=interpret,
    )(q, k, selq, selk)

    grid_spec = pltpu.PrefetchScalarGridSpec(
        num_scalar_prefetch=1,
        grid=(B, H, nbq // QPG),
        in_specs=[
            pl.BlockSpec((1, 1, QPG * BM, D), lambda b, h, qg, ids_r: (b, h, qg, 0)),
            pl.BlockSpec((1, 1, L, D), lambda b, h, qg, ids_r: (b, h, 0, 0)),
            pl.BlockSpec((1, 1, L, D), lambda b, h, qg, ids_r: (b, h, 0, 0)),
        ],
        out_specs=pl.BlockSpec((1, 1, QPG * BM, D), lambda b, h, qg, ids_r: (b, h, qg, 0)),
    )
    out = pl.pallas_call(
        _attn_kernel,
        grid_spec=grid_spec,
        out_shape=jax.ShapeDtypeStruct((B, H, L, D), jnp.float32),
        interpret=interpret,
    )(ids, q, k, v)
    return out


# speedup vs baseline: 5.0824x; 1.0047x over previous
import jax
import jax.numpy as jnp
from jax.experimental import pallas as pl
from jax.experimental.pallas import tpu as pltpu

BM = 128
BN = 64
NKM = BM // 4
NKN = BN // 4
LN2 = 0.6931471805599453
NEG = -1e30


def _mmx(a, b):
    return jax.lax.dot_general(a, b, (((1,), (0,)), ((), ())),
                               preferred_element_type=jnp.float32,
                               precision=jax.lax.Precision.HIGHEST)


def _mmd(a, b):
    return jax.lax.dot_general(a, b, (((1,), (0,)), ((), ())),
                               preferred_element_type=jnp.float32)


def _mmTd(a, b):
    return jax.lax.dot_general(a, b, (((1,), (1,)), ((), ())),
                               preferred_element_type=jnp.float32)


def _mask_kernel(q_ref, k_ref, selq_ref, selk_ref, ids_ref):
    L = q_ref.shape[2]
    nbq, nbk = L // BM, L // BN
    selq = selq_ref[0, 0]
    selk = selk_ref[0, 0]
    sq = jnp.concatenate(
        [_mmd(selq, q_ref[0, 0, i * BM:(i + 1) * BM, :]) for i in range(nbq)], 0)
    sk = jnp.concatenate(
        [_mmd(selk, k_ref[0, 0, j * BN:(j + 1) * BN, :]) for j in range(nbk)], 0)
    scale = 1.0 / (q_ref.shape[3] ** 0.5)
    logits = _mmTd(sq, sk) * scale
    m = jnp.max(logits, axis=1, keepdims=True)
    e = jnp.exp(logits - m)
    probs = e / jnp.sum(e, axis=1, keepdims=True)
    r = jax.lax.broadcasted_iota(jnp.int32, (nbq * NKM, nbk), 0)
    c = jax.lax.broadcasted_iota(jnp.int32, (nbq * NKM, nbk), 1)
    sumk = jnp.where(r // NKN == c, 1.0, 0.0)
    p_hi = probs.astype(jnp.bfloat16).astype(jnp.float32)
    r1 = probs - p_hi
    p_mid = r1.astype(jnp.bfloat16).astype(jnp.float32)
    p_lo = r1 - p_mid
    pk = _mmd(p_hi, sumk) + _mmd(p_mid, sumk) + _mmd(p_lo, sumk)
    r2 = jax.lax.broadcasted_iota(jnp.int32, (nbq * NKM, nbq), 0)
    c2 = jax.lax.broadcasted_iota(jnp.int32, (nbq * NKM, nbq), 1)
    meanq = jnp.where(r2 // NKM == c2, 1.0 / NKM, 0.0)
    pooling = jax.lax.dot_general(meanq, pk, (((0,), (0,)), ((), ())),
                                  preferred_element_type=jnp.float32,
                                  precision=jax.lax.Precision.HIGHEST)
    col = jax.lax.broadcasted_iota(jnp.int32, (nbq, nbk), 1)
    rank = jnp.zeros((nbq, nbk), jnp.int32)
    for jp in range(nbk):
        cv = pooling[:, jp:jp + 1]
        rank += (cv > pooling).astype(jnp.int32)
        rank += ((cv == pooling) & (jp < col)).astype(jnp.int32)
    ids = jnp.zeros((nbq, nbk), jnp.int32)
    for j in range(nbk):
        ids += j * (rank[:, j:j + 1] == col).astype(jnp.int32)
    ids_ref[0, 0] = ids


QPG = 16


def _attn_kernel(ids_ref, q_ref, k_ref, v_ref, o_ref):
    b = pl.program_id(0)
    h = pl.program_id(1)
    qg = pl.program_id(2)
    D = q_ref.shape[3]
    scale = 1.0 / (D ** 0.5)
    r8 = jax.lax.broadcasted_iota(jnp.int32, (8, BN), 0)
    c8 = jax.lax.broadcasted_iota(jnp.int32, (8, BN), 1)
    P8 = jnp.where(c8 // 8 == r8, 0.125, 0.0)
    r2 = jax.lax.broadcasted_iota(jnp.int32, (BN // 2, BN), 0)
    c2 = jax.lax.broadcasted_iota(jnp.int32, (BN // 2, BN), 1)
    P2 = jnp.where(c2 // 2 == r2, 0.5, 0.0)
    ci = jax.lax.broadcasted_iota(jnp.int32, (1, 384), 1)
    bias = jnp.where(ci < 256, 0.0,
                     jnp.where(ci < 288, LN2,
                               jnp.where(ci < 344, 3.0 * LN2, NEG)))
    zpad = jnp.zeros((40, BN), jnp.float32)

    for qq in range(QPG):
        qb = qg * QPG + qq
        q = q_ref[0, 0, qq * BM:(qq + 1) * BM, :]
        kbs = []
        vbs = []
        for s in range(12):
            bid = ids_ref[b, h, qb, s]
            kbs.append(k_ref[0, 0, pl.ds(bid * BN, BN), :])
            vbs.append(v_ref[0, 0, pl.ds(bid * BN, BN), :])
        ksel = jnp.concatenate(
            kbs[:4] + [_mmd(P2, kbs[4])] + [_mmd(P8, kbs[s]) for s in range(5, 12)]
            + [zpad], 0)
        vsel = jnp.concatenate(
            vbs[:4] + [_mmd(P2, vbs[4])] + [_mmd(P8, vbs[s]) for s in range(5, 12)]
            + [zpad], 0)
        s = _mmTd(q, ksel) * scale + bias
        m = jnp.max(s, axis=1, keepdims=True)
        e = jnp.exp(s - m)
        l = jnp.sum(e, axis=1, keepdims=True)
        o_ref[0, 0, qq * BM:(qq + 1) * BM, :] = _mmd(e, vsel) / l


def _sample_onehots(B, H):
    key = jax.random.key(42)
    k1, k2 = jax.random.split(key)
    rvq = jax.random.uniform(k1, (B, H, 1, BM))
    _, idxq = jax.lax.top_k(rvq, NKM)
    rvk = jax.random.uniform(k2, (B, H, 1, BN))
    _, idxk = jax.lax.top_k(rvk, NKN)
    selq = jax.nn.one_hot(idxq[:, :, 0, :], BM, dtype=jnp.float32)
    selk = jax.nn.one_hot(idxk[:, :, 0, :], BN, dtype=jnp.float32)
    return selq, selk


def kernel(q, k, v):
    B, H, L, D = q.shape
    selq, selk = _sample_onehots(B, H)
    nbq, nbk = L // BM, L // BN

    ids = pl.pallas_call(
        _mask_kernel,
        grid=(B, H),
        in_specs=[
            pl.BlockSpec((1, 1, L, D), lambda b, h: (b, h, 0, 0)),
            pl.BlockSpec((1, 1, L, D), lambda b, h: (b, h, 0, 0)),
            pl.BlockSpec((1, 1, NKM, BM), lambda b, h: (b, h, 0, 0)),
            pl.BlockSpec((1, 1, NKN, BN), lambda b, h: (b, h, 0, 0)),
        ],
        out_specs=pl.BlockSpec((1, 1, nbq, nbk), lambda b, h: (b, h, 0, 0)),
        out_shape=jax.ShapeDtypeStruct((B, H, nbq, nbk), jnp.int32),
    )(q, k, selq, selk)

    grid_spec = pltpu.PrefetchScalarGridSpec(
        num_scalar_prefetch=1,
        grid=(B, H, nbq // QPG),
        in_specs=[
            pl.BlockSpec((1, 1, QPG * BM, D), lambda b, h, qg, ids_r: (b, h, qg, 0)),
            pl.BlockSpec((1, 1, L, D), lambda b, h, qg, ids_r: (b, h, 0, 0)),
            pl.BlockSpec((1, 1, L, D), lambda b, h, qg, ids_r: (b, h, 0, 0)),
        ],
        out_specs=pl.BlockSpec((1, 1, QPG * BM, D), lambda b, h, qg, ids_r: (b, h, qg, 0)),
    )
    out = pl.pallas_call(
        _attn_kernel,
        grid_spec=grid_spec,
        out_shape=jax.ShapeDtypeStruct((B, H, L, D), jnp.float32),
    )(ids, q, k, v)
    return out
